# Initial kernel scaffold; baseline (speedup 1.0000x reference)
import functools

import jax
import jax.numpy as jnp
from jax import lax
from jax.experimental import pallas as pl
from jax.experimental.pallas import tpu as pltpu
from jax.experimental.pallas import tpu_sc as plsc

_N = 10000
_E = 320000
_D = 128
_G = 64

_NC = 2
_NS = 16
_NW = _NC * _NS
_EPW = _E // _NW
_CH = 80
_NCH = _EPW // _CH
_RPS = _N // _NS
_ZR = 125

_ROWB = 1000


def _sc_mesh():
    return plsc.VectorSubcoreMesh(core_axis_name="c", subcore_axis_name="s")


def _sc_degree(dst):
    @functools.partial(
        pl.kernel,
        out_type=jax.ShapeDtypeStruct((2 * _N, 16), jnp.float32),
        mesh=_sc_mesh(),
        scratch_types=[
            pltpu.VMEM((_CH,), jnp.int32),
            pltpu.VMEM((_CH, 16), jnp.float32),
            pltpu.VMEM((_ZR, 16), jnp.float32),
            pltpu.VMEM_SHARED((_N, 16), jnp.float32),
        ],
    )
    def k(dst_hbm, out_hbm, idx_v, ones_v, zer_v, acc_sh):
        c = lax.axis_index("c")
        s = lax.axis_index("s")
        zv = jnp.zeros((16,), jnp.float32)
        ov = jnp.ones((16,), jnp.float32)

        @pl.loop(0, _ZR)
        def _(i):
            zer_v[i, :] = zv

        @pl.loop(0, _CH)
        def _(i):
            ones_v[i, :] = ov

        r0 = s * _RPS

        @pl.loop(0, _RPS // _ZR)
        def _(j):
            pltpu.sync_copy(zer_v, acc_sh.at[pl.ds(r0 + j * _ZR, _ZR)])

        plsc.subcore_barrier()
        base = (c * _NS + s) * _EPW

        @pl.loop(0, _NCH)
        def _(kk):
            pltpu.sync_copy(dst_hbm.at[pl.ds(base + kk * _CH, _CH)], idx_v)
            pltpu.sync_copy(ones_v, acc_sh.at[idx_v], add=True)

        plsc.subcore_barrier()
        pltpu.sync_copy(acc_sh.at[pl.ds(r0, _RPS)],
                        out_hbm.at[pl.ds(c * _N + r0, _RPS)])

    return k(dst)


def _sc_scatter(m, src, dst):
    @functools.partial(
        pl.kernel,
        out_type=jax.ShapeDtypeStruct((2 * _N, _D), jnp.float32),
        mesh=_sc_mesh(),
        scratch_types=[
            pltpu.VMEM((_CH,), jnp.int32),
            pltpu.VMEM((_CH,), jnp.int32),
            pltpu.VMEM((_CH, _D), jnp.float32),
            pltpu.VMEM((_ZR, _D), jnp.float32),
            pltpu.VMEM_SHARED((_N, _D), jnp.float32),
            pltpu.SemaphoreType.DMA,
        ],
    )
    def k(m_hbm, src_hbm, dst_hbm, out_hbm, gi_v, di_v, rows_v, zer_v,
          acc_sh, sem):
        c = lax.axis_index("c")
        s = lax.axis_index("s")
        zv = jnp.zeros((16,), jnp.float32)

        @pl.loop(0, _ZR)
        def _(i):
            @pl.loop(0, _D // 16)
            def _(j):
                zer_v[i, pl.ds(j * 16, 16)] = zv

        r0 = s * _RPS

        @pl.loop(0, _RPS // _ZR)
        def _(j):
            pltpu.sync_copy(zer_v, acc_sh.at[pl.ds(r0 + j * _ZR, _ZR)])

        plsc.subcore_barrier()
        base = (c * _NS + s) * _EPW

        @pl.loop(0, _NCH)
        def _(kk):
            off = base + kk * _CH
            pltpu.sync_copy(src_hbm.at[pl.ds(off, _CH)], gi_v)
            pltpu.sync_copy(dst_hbm.at[pl.ds(off, _CH)], di_v)
            pltpu.async_copy(m_hbm.at[gi_v], rows_v, sem).wait()
            pltpu.sync_copy(rows_v, acc_sh.at[di_v], add=True)

        plsc.subcore_barrier()
        pltpu.sync_copy(acc_sh.at[pl.ds(r0, _RPS)],
                        out_hbm.at[pl.ds(c * _N + r0, _RPS)])

    return k(m, src, dst)


def _row_spec(i_map=lambda i: (i, 0), shape=(_ROWB, _D)):
    return pl.BlockSpec(shape, i_map)


def _tc_matmul(x, w):
    def body(x_ref, w_ref, o_ref):
        o_ref[...] = jnp.dot(x_ref[...], w_ref[...],
                             preferred_element_type=jnp.float32)

    return pl.pallas_call(
        body,
        grid=(_N // _ROWB,),
        in_specs=[_row_spec(), pl.BlockSpec((_D, _D), lambda i: (0, 0))],
        out_specs=_row_spec(),
        out_shape=jax.ShapeDtypeStruct((_N, _D), jnp.float32),
    )(x, w)


def _tc_scale(m1, degp):
    def body(m_ref, d0_ref, d1_ref, mp_ref, dinv_ref):
        deg = 1.0 + d0_ref[:, 0:1] + d1_ref[:, 0:1]
        dinv = lax.rsqrt(deg)
        dinv_ref[...] = dinv
        mp_ref[...] = m_ref[...] * dinv

    return pl.pallas_call(
        body,
        grid=(_N // _ROWB,),
        in_specs=[
            _row_spec(),
            _row_spec(lambda i: (i, 0), (_ROWB, 16)),
            _row_spec(lambda i: (i + _N // _ROWB, 0), (_ROWB, 16)),
        ],
        out_specs=(_row_spec(), _row_spec(lambda i: (i, 0), (_ROWB, 1))),
        out_shape=(jax.ShapeDtypeStruct((_N, _D), jnp.float32),
                   jax.ShapeDtypeStruct((_N, 1), jnp.float32)),
    )(m1, degp, degp)


def _tc_mid(S, mp, dinv, b, w):
    def body(sa_ref, sb_ref, mp_ref, dinv_ref, b_ref, w_ref, o_ref):
        dv = dinv_ref[...]
        h = dv * (sa_ref[...] + sb_ref[...] + mp_ref[...]) + b_ref[...]
        h = jnp.maximum(h, 0.0)
        o_ref[...] = dv * jnp.dot(h, w_ref[...],
                                  preferred_element_type=jnp.float32)

    nb = _N // _ROWB
    return pl.pallas_call(
        body,
        grid=(nb,),
        in_specs=[
            _row_spec(),
            _row_spec(lambda i: (i + nb, 0)),
            _row_spec(),
            _row_spec(lambda i: (i, 0), (_ROWB, 1)),
            pl.BlockSpec((1, _D), lambda i: (0, 0)),
            pl.BlockSpec((_D, _D), lambda i: (0, 0)),
        ],
        out_specs=_row_spec(),
        out_shape=jax.ShapeDtypeStruct((_N, _D), jnp.float32),
    )(S, S, mp, dinv, b, w)


def _tc_final(S, mp, dinv, b, batch3, wf1, bf1, wf2, bf2):
    nb = _N // _ROWB

    def body(sa_ref, sb_ref, mp_ref, dinv_ref, b_ref, bat_ref, wf1_ref,
             bf1_ref, wf2_ref, bf2_ref, o_ref, sums, cnt):
        i = pl.program_id(0)

        @pl.when(i == 0)
        def _():
            sums[...] = jnp.zeros((_G, _D), jnp.float32)
            cnt[...] = jnp.zeros((_G, 1), jnp.float32)

        dv = dinv_ref[...]
        h = dv * (sa_ref[...] + sb_ref[...] + mp_ref[...]) + b_ref[...]
        h = jnp.maximum(h, 0.0)
        ids = lax.broadcasted_iota(jnp.int32, (_G, _ROWB), 0)
        oh = jnp.where(bat_ref[0] == ids, 1.0, 0.0)
        sums[...] += jnp.dot(oh, h, preferred_element_type=jnp.float32)
        cnt[...] += jnp.sum(oh, axis=1, keepdims=True)

        @pl.when(i == nb - 1)
        def _():
            pooled = sums[...] / jnp.maximum(cnt[...], 1.0)
            z = jnp.dot(pooled, wf1_ref[...],
                        preferred_element_type=jnp.float32) + bf1_ref[...]
            z = jnp.maximum(z, 0.0)
            z = jnp.dot(z, wf2_ref[...],
                        preferred_element_type=jnp.float32) + bf2_ref[...]
            o_ref[...] = 1.0 / (1.0 + jnp.exp(-z))

    return pl.pallas_call(
        body,
        grid=(nb,),
        in_specs=[
            _row_spec(),
            _row_spec(lambda i: (i + nb, 0)),
            _row_spec(),
            _row_spec(lambda i: (i, 0), (_ROWB, 1)),
            pl.BlockSpec((1, _D), lambda i: (0, 0)),
            pl.BlockSpec((1, 1, _ROWB), lambda i: (i, 0, 0)),
            pl.BlockSpec((_D, _D), lambda i: (0, 0)),
            pl.BlockSpec((1, _D), lambda i: (0, 0)),
            pl.BlockSpec((_D, 1), lambda i: (0, 0)),
            pl.BlockSpec((1, 1), lambda i: (0, 0)),
        ],
        out_specs=pl.BlockSpec((_G, 1), lambda i: (0, 0)),
        out_shape=jax.ShapeDtypeStruct((_G, 1), jnp.float32),
        scratch_shapes=[pltpu.VMEM((_G, _D), jnp.float32),
                        pltpu.VMEM((_G, 1), jnp.float32)],
    )(S, S, mp, dinv, b, batch3, wf1, bf1, wf2, bf2)


def kernel(x, edge_index, batch, W1, b1, W2, b2, Wf1, bf1, Wf2, bf2):
    src = edge_index[0]
    dst = edge_index[1]
    b1r = b1.reshape(1, _D)
    b2r = b2.reshape(1, _D)
    bf1r = bf1.reshape(1, _D)
    bf2r = bf2.reshape(1, 1)
    batch3 = batch.reshape(_N // _ROWB, 1, _ROWB)

    degp = _sc_degree(dst)
    m1 = _tc_matmul(x, W1)
    m1p, dinv = _tc_scale(m1, degp)
    S1 = _sc_scatter(m1p, src, dst)
    m2p = _tc_mid(S1, m1p, dinv, b1r, W2)
    S2 = _sc_scatter(m2p, src, dst)
    return _tc_final(S2, m2p, dinv, b2r, batch3, Wf1, bf1r, Wf2, bf2r)


# speedup vs baseline: 12.8277x; 12.8277x over previous
import functools

import jax
import jax.numpy as jnp
from jax import lax
from jax.experimental import pallas as pl
from jax.experimental.pallas import tpu as pltpu
from jax.experimental.pallas import tpu_sc as plsc

_N = 10000
_E = 320000
_D = 128
_G = 64

_NC = 2
_NS = 16
_NW = _NC * _NS
_EPW = _E // _NW
_CH = 80
_NCH = _EPW // _CH
_RPS = _N // _NS
_ZR = 125

_ROWB = 1000


def _sc_mesh():
    return plsc.VectorSubcoreMesh(core_axis_name="c", subcore_axis_name="s")


def _sc_degree(dst):
    @functools.partial(
        pl.kernel,
        out_type=jax.ShapeDtypeStruct((_NC, _NS, _RPS, 16), jnp.float32),
        mesh=_sc_mesh(),
        scratch_types=[
            pltpu.VMEM((_CH,), jnp.int32),
            pltpu.VMEM((_CH, 16), jnp.float32),
            pltpu.VMEM((_ZR, 16), jnp.float32),
            pltpu.VMEM_SHARED((_N, 16), jnp.float32),
        ],
    )
    def k(dst_hbm, out_hbm, idx_v, ones_v, zer_v, acc_sh):
        c = lax.axis_index("c")
        s = lax.axis_index("s")
        zv = jnp.zeros((16,), jnp.float32)
        ov = jnp.ones((16,), jnp.float32)

        @pl.loop(0, _ZR)
        def _(i):
            zer_v[i, :] = zv

        @pl.loop(0, _CH)
        def _(i):
            ones_v[i, :] = ov

        r0 = s * _RPS

        @pl.loop(0, _RPS // _ZR)
        def _(j):
            pltpu.sync_copy(zer_v, acc_sh.at[pl.ds(r0 + j * _ZR, _ZR)])

        plsc.subcore_barrier()
        base = (c * _NS + s) * _EPW

        @pl.loop(0, _NCH)
        def _(kk):
            pltpu.sync_copy(dst_hbm.at[pl.ds(base + kk * _CH, _CH)], idx_v)
            pltpu.sync_copy(ones_v, acc_sh.at[idx_v], add=True)

        plsc.subcore_barrier()
        pltpu.sync_copy(acc_sh.at[pl.ds(r0, _RPS)], out_hbm.at[c, s])

    return k(dst).reshape(2 * _N, 16)


def _sc_scatter(m, src, dst):
    @functools.partial(
        pl.kernel,
        out_type=jax.ShapeDtypeStruct((_NC, _NS, _RPS, _D), jnp.float32),
        mesh=_sc_mesh(),
        scratch_types=[
            pltpu.VMEM((_CH,), jnp.int32),
            pltpu.VMEM((_CH,), jnp.int32),
            pltpu.VMEM((_CH, _D), jnp.float32),
            pltpu.VMEM((_ZR, _D), jnp.float32),
            pltpu.VMEM_SHARED((_N, _D), jnp.float32),
            pltpu.SemaphoreType.DMA,
        ],
    )
    def k(m_hbm, src_hbm, dst_hbm, out_hbm, gi_v, di_v, rows_v, zer_v,
          acc_sh, sem):
        c = lax.axis_index("c")
        s = lax.axis_index("s")
        zv = jnp.zeros((16,), jnp.float32)

        @pl.loop(0, _ZR)
        def _(i):
            @pl.loop(0, _D // 16)
            def _(j):
                zer_v[i, pl.ds(j * 16, 16)] = zv

        r0 = s * _RPS

        @pl.loop(0, _RPS // _ZR)
        def _(j):
            pltpu.sync_copy(zer_v, acc_sh.at[pl.ds(r0 + j * _ZR, _ZR)])

        plsc.subcore_barrier()
        base = (c * _NS + s) * _EPW

        @pl.loop(0, _NCH)
        def _(kk):
            off = base + kk * _CH
            pltpu.sync_copy(src_hbm.at[pl.ds(off, _CH)], gi_v)
            pltpu.sync_copy(dst_hbm.at[pl.ds(off, _CH)], di_v)
            pltpu.async_copy(m_hbm.at[gi_v], rows_v, sem).wait()
            pltpu.sync_copy(rows_v, acc_sh.at[di_v], add=True)

        plsc.subcore_barrier()
        pltpu.sync_copy(acc_sh.at[pl.ds(r0, _RPS)], out_hbm.at[c, s])

    return k(m, src, dst).reshape(2 * _N, _D)


def _row_spec(i_map=lambda i: (i, 0), shape=(_ROWB, _D)):
    return pl.BlockSpec(shape, i_map)


def _tc_matmul(x, w):
    def body(x_ref, w_ref, o_ref):
        o_ref[...] = jnp.dot(x_ref[...], w_ref[...],
                             preferred_element_type=jnp.float32)

    return pl.pallas_call(
        body,
        grid=(_N // _ROWB,),
        in_specs=[_row_spec(), pl.BlockSpec((_D, _D), lambda i: (0, 0))],
        out_specs=_row_spec(),
        out_shape=jax.ShapeDtypeStruct((_N, _D), jnp.float32),
    )(x, w)


def _tc_scale(m1, degp):
    def body(m_ref, d0_ref, d1_ref, mp_ref, dinv_ref):
        deg = 1.0 + d0_ref[:, 0:1] + d1_ref[:, 0:1]
        dinv = lax.rsqrt(deg)
        dinv_ref[...] = dinv
        mp_ref[...] = m_ref[...] * dinv

    return pl.pallas_call(
        body,
        grid=(_N // _ROWB,),
        in_specs=[
            _row_spec(),
            _row_spec(lambda i: (i, 0), (_ROWB, 16)),
            _row_spec(lambda i: (i + _N // _ROWB, 0), (_ROWB, 16)),
        ],
        out_specs=(_row_spec(), _row_spec(lambda i: (i, 0), (_ROWB, 1))),
        out_shape=(jax.ShapeDtypeStruct((_N, _D), jnp.float32),
                   jax.ShapeDtypeStruct((_N, 1), jnp.float32)),
    )(m1, degp, degp)


def _tc_mid(S, mp, dinv, b, w):
    def body(sa_ref, sb_ref, mp_ref, dinv_ref, b_ref, w_ref, o_ref):
        dv = dinv_ref[...]
        h = dv * (sa_ref[...] + sb_ref[...] + mp_ref[...]) + b_ref[...]
        h = jnp.maximum(h, 0.0)
        o_ref[...] = dv * jnp.dot(h, w_ref[...],
                                  preferred_element_type=jnp.float32)

    nb = _N // _ROWB
    return pl.pallas_call(
        body,
        grid=(nb,),
        in_specs=[
            _row_spec(),
            _row_spec(lambda i: (i + nb, 0)),
            _row_spec(),
            _row_spec(lambda i: (i, 0), (_ROWB, 1)),
            pl.BlockSpec((1, _D), lambda i: (0, 0)),
            pl.BlockSpec((_D, _D), lambda i: (0, 0)),
        ],
        out_specs=_row_spec(),
        out_shape=jax.ShapeDtypeStruct((_N, _D), jnp.float32),
    )(S, S, mp, dinv, b, w)


def _tc_final(S, mp, dinv, b, batch3, wf1, bf1, wf2, bf2):
    nb = _N // _ROWB

    def body(sa_ref, sb_ref, mp_ref, dinv_ref, b_ref, bat_ref, wf1_ref,
             bf1_ref, wf2_ref, bf2_ref, o_ref, sums, cnt):
        i = pl.program_id(0)

        @pl.when(i == 0)
        def _():
            sums[...] = jnp.zeros((_G, _D), jnp.float32)
            cnt[...] = jnp.zeros((_G, 1), jnp.float32)

        dv = dinv_ref[...]
        h = dv * (sa_ref[...] + sb_ref[...] + mp_ref[...]) + b_ref[...]
        h = jnp.maximum(h, 0.0)
        ids = lax.broadcasted_iota(jnp.int32, (_G, _ROWB), 0)
        oh = jnp.where(bat_ref[0] == ids, 1.0, 0.0)
        sums[...] += jnp.dot(oh, h, preferred_element_type=jnp.float32)
        cnt[...] += jnp.sum(oh, axis=1, keepdims=True)

        @pl.when(i == nb - 1)
        def _():
            pooled = sums[...] / jnp.maximum(cnt[...], 1.0)
            z = jnp.dot(pooled, wf1_ref[...],
                        preferred_element_type=jnp.float32) + bf1_ref[...]
            z = jnp.maximum(z, 0.0)
            z = jnp.dot(z, wf2_ref[...],
                        preferred_element_type=jnp.float32) + bf2_ref[...]
            o_ref[...] = 1.0 / (1.0 + jnp.exp(-z))

    return pl.pallas_call(
        body,
        grid=(nb,),
        in_specs=[
            _row_spec(),
            _row_spec(lambda i: (i + nb, 0)),
            _row_spec(),
            _row_spec(lambda i: (i, 0), (_ROWB, 1)),
            pl.BlockSpec((1, _D), lambda i: (0, 0)),
            pl.BlockSpec((1, 1, _ROWB), lambda i: (i, 0, 0)),
            pl.BlockSpec((_D, _D), lambda i: (0, 0)),
            pl.BlockSpec((1, _D), lambda i: (0, 0)),
            pl.BlockSpec((_D, 1), lambda i: (0, 0)),
            pl.BlockSpec((1, 1), lambda i: (0, 0)),
        ],
        out_specs=pl.BlockSpec((_G, 1), lambda i: (0, 0)),
        out_shape=jax.ShapeDtypeStruct((_G, 1), jnp.float32),
        scratch_shapes=[pltpu.VMEM((_G, _D), jnp.float32),
                        pltpu.VMEM((_G, 1), jnp.float32)],
    )(S, S, mp, dinv, b, batch3, wf1, bf1, wf2, bf2)


def kernel(x, edge_index, batch, W1, b1, W2, b2, Wf1, bf1, Wf2, bf2):
    src = edge_index[0]
    dst = edge_index[1]
    b1r = b1.reshape(1, _D)
    b2r = b2.reshape(1, _D)
    bf1r = bf1.reshape(1, _D)
    bf2r = bf2.reshape(1, 1)
    batch3 = batch.reshape(_N // _ROWB, 1, _ROWB)

    degp = _sc_degree(dst)
    m1 = _tc_matmul(x, W1)
    m1p, dinv = _tc_scale(m1, degp)
    S1 = _sc_scatter(m1p, src, dst)
    m2p = _tc_mid(S1, m1p, dinv, b1r, W2)
    S2 = _sc_scatter(m2p, src, dst)
    return _tc_final(S2, m2p, dinv, b2r, batch3, Wf1, bf1r, Wf2, bf2r)


# speedup vs baseline: 32.5815x; 2.5399x over previous
import functools

import jax
import jax.numpy as jnp
from jax import lax
from jax.experimental import pallas as pl
from jax.experimental.pallas import tpu as pltpu
from jax.experimental.pallas import tpu_sc as plsc

_N = 10000
_E = 320000
_D = 128
_G = 64

_NC = 2
_NS = 16
_NW = _NC * _NS
_EPW = _E // _NW
_CH = 80
_NCH = _EPW // _CH
_RPS = _N // _NS
_ZR = 125

_ROWB = 1000


def _sc_mesh():
    return plsc.VectorSubcoreMesh(core_axis_name="c", subcore_axis_name="s")


def _sc_degree(dst):
    @functools.partial(
        pl.kernel,
        out_type=jax.ShapeDtypeStruct((_NC, _NS, _RPS, 16), jnp.float32),
        mesh=_sc_mesh(),
        scratch_types=(
            [pltpu.VMEM((_CH,), jnp.int32)] * _NBUF
            + [pltpu.VMEM((_CH, 16), jnp.float32)]
            + [pltpu.VMEM((_ZR, 16), jnp.float32)]
            + [pltpu.VMEM_SHARED((_N, 16), jnp.float32)]
            + [pltpu.SemaphoreType.DMA] * (2 * _NBUF)
        ),
    )
    def k(dst_hbm, out_hbm, *refs):
        di = refs[0:_NBUF]
        ones_v = refs[_NBUF]
        zer_v = refs[_NBUF + 1]
        acc_sh = refs[_NBUF + 2]
        lsems = refs[_NBUF + 3:_NBUF + 3 + _NBUF]
        ssems = refs[_NBUF + 3 + _NBUF:]
        c = lax.axis_index("c")
        s = lax.axis_index("s")
        base = (c * _NS + s) * _EPW
        zv = jnp.zeros((16,), jnp.float32)
        ov = jnp.ones((16,), jnp.float32)

        def issue_load(b, kk):
            pltpu.async_copy(dst_hbm.at[pl.ds(base + kk * _CH, _CH)], di[b],
                             lsems[b])

        def wait_load(b, kk):
            pltpu.make_async_copy(dst_hbm.at[pl.ds(base + kk * _CH, _CH)],
                                  di[b], lsems[b]).wait()

        def issue_scatter(b):
            pltpu.async_copy(ones_v, acc_sh.at[di[b]], ssems[b], add=True)

        def wait_scatter(b):
            pltpu.make_async_copy(ones_v, acc_sh.at[di[b]], ssems[b]).wait()

        for b in range(_NBUF):
            issue_load(b, b)

        @pl.loop(0, _ZR)
        def _(i):
            zer_v[i, :] = zv

        @pl.loop(0, _CH)
        def _(i):
            ones_v[i, :] = ov

        r0 = s * _RPS

        @pl.loop(0, _RPS // _ZR)
        def _(j):
            pltpu.sync_copy(zer_v, acc_sh.at[pl.ds(r0 + j * _ZR, _ZR)])

        plsc.subcore_barrier()

        @pl.loop(0, _NCH // _NBUF)
        def _(t):
            for b in range(_NBUF):
                kk = t * _NBUF + b
                wait_load(b, kk)
                issue_scatter(b)
                wait_scatter(b)

                @pl.when(kk + _NBUF < _NCH)
                def _():
                    issue_load(b, kk + _NBUF)

        wait_load(0, _NCH - 1)
        issue_scatter(0)
        wait_scatter(0)

        plsc.subcore_barrier()
        pltpu.sync_copy(acc_sh.at[pl.ds(r0, _RPS)], out_hbm.at[c, s])

    return k(dst).reshape(2 * _N, 16)


_NBUF = 4
_GA = 2
_ZRS = 25


def _sc_scatter(m, src, dst):
    @functools.partial(
        pl.kernel,
        out_type=jax.ShapeDtypeStruct((_NC, _NS, _RPS, _D), jnp.float32),
        mesh=_sc_mesh(),
        scratch_types=(
            [pltpu.VMEM((_CH,), jnp.int32)] * _NBUF
            + [pltpu.VMEM((_CH,), jnp.int32)] * _NBUF
            + [pltpu.VMEM((_CH, _D), jnp.float32)] * _NBUF
            + [pltpu.VMEM((_ZRS, _D), jnp.float32)]
            + [pltpu.VMEM_SHARED((_N, _D), jnp.float32)]
            + [pltpu.SemaphoreType.DMA] * (2 * _NBUF)
        ),
    )
    def k(m_hbm, src_hbm, dst_hbm, out_hbm, *refs):
        gi = refs[0:_NBUF]
        di = refs[_NBUF:2 * _NBUF]
        rows = refs[2 * _NBUF:3 * _NBUF]
        zer_v = refs[3 * _NBUF]
        acc_sh = refs[3 * _NBUF + 1]
        lsems = refs[3 * _NBUF + 2:3 * _NBUF + 2 + _NBUF]
        gsems = refs[3 * _NBUF + 2 + _NBUF:]
        c = lax.axis_index("c")
        s = lax.axis_index("s")
        base = (c * _NS + s) * _EPW

        def issue_loads(b, kk):
            off = base + kk * _CH
            pltpu.async_copy(src_hbm.at[pl.ds(off, _CH)], gi[b], lsems[b])
            pltpu.async_copy(dst_hbm.at[pl.ds(off, _CH)], di[b], lsems[b])

        def wait_loads(b, kk):
            off = base + kk * _CH
            pltpu.make_async_copy(src_hbm.at[pl.ds(off, _CH)], gi[b],
                                  lsems[b]).wait()
            pltpu.make_async_copy(dst_hbm.at[pl.ds(off, _CH)], di[b],
                                  lsems[b]).wait()

        def issue_gather(b):
            pltpu.async_copy(m_hbm.at[gi[b]], rows[b], gsems[b])

        def wait_gather(b):
            pltpu.make_async_copy(m_hbm.at[gi[b]], rows[b], gsems[b]).wait()

        def scatter(b):
            pltpu.sync_copy(rows[b], acc_sh.at[di[b]], add=True)

        for b in range(_NBUF):
            issue_loads(b, b)

        zv = jnp.zeros((16,), jnp.float32)

        @pl.loop(0, _ZRS)
        def _(i):
            @pl.loop(0, _D // 16)
            def _(j):
                zer_v[i, pl.ds(j * 16, 16)] = zv

        r0 = s * _RPS

        @pl.loop(0, _RPS // _ZRS)
        def _(j):
            pltpu.sync_copy(zer_v, acc_sh.at[pl.ds(r0 + j * _ZRS, _ZRS)])

        plsc.subcore_barrier()
        for b in range(_GA):
            wait_loads(b, b)
            issue_gather(b)

        @pl.loop(0, _NCH // _NBUF)
        def _(t):
            for b in range(_NBUF):
                kk = t * _NBUF + b
                b2 = (b + _GA) % _NBUF

                @pl.when(kk + _GA < _NCH)
                def _():
                    wait_loads(b2, kk + _GA)
                    issue_gather(b2)

                wait_gather(b)
                scatter(b)

                @pl.when(kk + _NBUF < _NCH)
                def _():
                    issue_loads(b, kk + _NBUF)

        wait_gather(0)
        scatter(0)

        plsc.subcore_barrier()
        pltpu.sync_copy(acc_sh.at[pl.ds(r0, _RPS)], out_hbm.at[c, s])

    return k(m, src, dst).reshape(2 * _N, _D)


def _row_spec(i_map=lambda i: (i, 0), shape=(_ROWB, _D)):
    return pl.BlockSpec(shape, i_map)


def _tc_matmul(x, w):
    def body(x_ref, w_ref, o_ref):
        o_ref[...] = jnp.dot(x_ref[...], w_ref[...],
                             preferred_element_type=jnp.float32)

    return pl.pallas_call(
        body,
        grid=(_N // _ROWB,),
        in_specs=[_row_spec(), pl.BlockSpec((_D, _D), lambda i: (0, 0))],
        out_specs=_row_spec(),
        out_shape=jax.ShapeDtypeStruct((_N, _D), jnp.float32),
    )(x, w)


def _tc_scale(m1, degp):
    def body(m_ref, d0_ref, d1_ref, mp_ref, dinv_ref):
        deg = 1.0 + d0_ref[:, 0:1] + d1_ref[:, 0:1]
        dinv = lax.rsqrt(deg)
        dinv_ref[...] = dinv
        mp_ref[...] = m_ref[...] * dinv

    return pl.pallas_call(
        body,
        grid=(_N // _ROWB,),
        in_specs=[
            _row_spec(),
            _row_spec(lambda i: (i, 0), (_ROWB, 16)),
            _row_spec(lambda i: (i + _N // _ROWB, 0), (_ROWB, 16)),
        ],
        out_specs=(_row_spec(), _row_spec(lambda i: (i, 0), (_ROWB, 1))),
        out_shape=(jax.ShapeDtypeStruct((_N, _D), jnp.float32),
                   jax.ShapeDtypeStruct((_N, 1), jnp.float32)),
    )(m1, degp, degp)


def _tc_mid(S, mp, dinv, b, w):
    def body(sa_ref, sb_ref, mp_ref, dinv_ref, b_ref, w_ref, o_ref):
        dv = dinv_ref[...]
        h = dv * (sa_ref[...] + sb_ref[...] + mp_ref[...]) + b_ref[...]
        h = jnp.maximum(h, 0.0)
        o_ref[...] = dv * jnp.dot(h, w_ref[...],
                                  preferred_element_type=jnp.float32)

    nb = _N // _ROWB
    return pl.pallas_call(
        body,
        grid=(nb,),
        in_specs=[
            _row_spec(),
            _row_spec(lambda i: (i + nb, 0)),
            _row_spec(),
            _row_spec(lambda i: (i, 0), (_ROWB, 1)),
            pl.BlockSpec((1, _D), lambda i: (0, 0)),
            pl.BlockSpec((_D, _D), lambda i: (0, 0)),
        ],
        out_specs=_row_spec(),
        out_shape=jax.ShapeDtypeStruct((_N, _D), jnp.float32),
    )(S, S, mp, dinv, b, w)


def _tc_final(S, mp, dinv, b, batch3, wf1, bf1, wf2, bf2):
    nb = _N // _ROWB

    def body(sa_ref, sb_ref, mp_ref, dinv_ref, b_ref, bat_ref, wf1_ref,
             bf1_ref, wf2_ref, bf2_ref, o_ref, sums, cnt):
        i = pl.program_id(0)

        @pl.when(i == 0)
        def _():
            sums[...] = jnp.zeros((_G, _D), jnp.float32)
            cnt[...] = jnp.zeros((_G, 1), jnp.float32)

        dv = dinv_ref[...]
        h = dv * (sa_ref[...] + sb_ref[...] + mp_ref[...]) + b_ref[...]
        h = jnp.maximum(h, 0.0)
        ids = lax.broadcasted_iota(jnp.int32, (_G, _ROWB), 0)
        oh = jnp.where(bat_ref[0] == ids, 1.0, 0.0)
        sums[...] += jnp.dot(oh, h, preferred_element_type=jnp.float32)
        cnt[...] += jnp.sum(oh, axis=1, keepdims=True)

        @pl.when(i == nb - 1)
        def _():
            pooled = sums[...] / jnp.maximum(cnt[...], 1.0)
            z = jnp.dot(pooled, wf1_ref[...],
                        preferred_element_type=jnp.float32) + bf1_ref[...]
            z = jnp.maximum(z, 0.0)
            z = jnp.dot(z, wf2_ref[...],
                        preferred_element_type=jnp.float32) + bf2_ref[...]
            o_ref[...] = 1.0 / (1.0 + jnp.exp(-z))

    return pl.pallas_call(
        body,
        grid=(nb,),
        in_specs=[
            _row_spec(),
            _row_spec(lambda i: (i + nb, 0)),
            _row_spec(),
            _row_spec(lambda i: (i, 0), (_ROWB, 1)),
            pl.BlockSpec((1, _D), lambda i: (0, 0)),
            pl.BlockSpec((1, 1, _ROWB), lambda i: (i, 0, 0)),
            pl.BlockSpec((_D, _D), lambda i: (0, 0)),
            pl.BlockSpec((1, _D), lambda i: (0, 0)),
            pl.BlockSpec((_D, 1), lambda i: (0, 0)),
            pl.BlockSpec((1, 1), lambda i: (0, 0)),
        ],
        out_specs=pl.BlockSpec((_G, 1), lambda i: (0, 0)),
        out_shape=jax.ShapeDtypeStruct((_G, 1), jnp.float32),
        scratch_shapes=[pltpu.VMEM((_G, _D), jnp.float32),
                        pltpu.VMEM((_G, 1), jnp.float32)],
    )(S, S, mp, dinv, b, batch3, wf1, bf1, wf2, bf2)


def kernel(x, edge_index, batch, W1, b1, W2, b2, Wf1, bf1, Wf2, bf2):
    src = edge_index[0]
    dst = edge_index[1]
    b1r = b1.reshape(1, _D)
    b2r = b2.reshape(1, _D)
    bf1r = bf1.reshape(1, _D)
    bf2r = bf2.reshape(1, 1)
    batch3 = batch.reshape(_N // _ROWB, 1, _ROWB)

    degp = _sc_degree(dst)
    m1 = _tc_matmul(x, W1)
    m1p, dinv = _tc_scale(m1, degp)
    S1 = _sc_scatter(m1p, src, dst)
    m2p = _tc_mid(S1, m1p, dinv, b1r, W2)
    S2 = _sc_scatter(m2p, src, dst)
    return _tc_final(S2, m2p, dinv, b2r, batch3, Wf1, bf1r, Wf2, bf2r)
